# baseline (speedup 1.0000x reference)
import jax
import jax.numpy as jnp
from jax import lax
from jax.experimental import pallas as pl
from jax.experimental.pallas import tpu as pltpu
from jax.experimental.pallas import tpu_sc as plsc

B = 4096
L = 50
D = 32
NUM_CORES = 2
NUM_SUBCORES = 16
NW = NUM_CORES * NUM_SUBCORES
BPW = B // NW
TPW = BPW * L
GROUP = 128 // D
SC_ROWS = 8
SC_TOK = SC_ROWS * L
NSC = BPW // SC_ROWS
GCH = 40
GPS = SC_TOK // GCH
HALF = 16
IDX_COLS = 64
IDX_ROWS = TPW // IDX_COLS


def _sc_body(table_ref, idx_ref, out_ref, idx_v, gidx_v, colb_v,
             buf0, buf1, out_v, sem0, sem1):
    wid = lax.axis_index("s") * NUM_CORES + lax.axis_index("c")

    pltpu.sync_copy(idx_ref.at[wid], idx_v)

    def prep(k, carry):
        j = k >> 2
        c = (k & 3) * HALF
        v = idx_v[j, pl.ds(c, HALF)]
        t = k * HALF
        gidx_v[pl.ds(t, HALF)] = lax.shift_right_logical(v, 2)
        colb_v[pl.ds(t, HALF)] = lax.shift_left(v & 3, 5)
        return carry

    lax.fori_loop(0, TPW // HALF, prep, 0)

    bufs = (buf0, buf1)
    sems = (sem0, sem1)

    def fire(sc):
        buf = bufs[sc % 2]
        sem = sems[sc % 2]
        handles = []
        for m in range(GPS):
            off = sc * SC_TOK + m * GCH
            h = pltpu.async_copy(
                table_ref.at[gidx_v.at[pl.ds(off, GCH)]],
                buf.at[pl.ds(m * GCH, GCH)],
                sem,
            )
            handles.append(h)
        return handles

    inv_l = jnp.float32(1.0 / L)

    def reduce_chunk(sc):
        buf = bufs[sc % 2]
        t0 = sc * SC_TOK

        def body(b, carry):
            p0 = b * L
            base = t0 + p0
            cvecs = [colb_v[pl.ds(base + HALF * i, HALF)]
                     for i in range((L + HALF - 1) // HALF + 1)]
            cbs = [cvecs[l // HALF][l % HALF] for l in range(L)]
            for h in range(2):
                parts = [buf[p0 + k, pl.ds(cbs[k] + h * HALF, HALF)]
                         for k in range(4)]
                for l in range(4, L):
                    parts[l % 4] = parts[l % 4] + buf[
                        p0 + l, pl.ds(cbs[l] + h * HALF, HALF)]
                s = (parts[0] + parts[1]) + (parts[2] + parts[3])
                out_v[b, pl.ds(h * HALF, HALF)] = s * inv_l
            return carry

        lax.fori_loop(0, SC_ROWS, body, 0)
        row0 = wid * BPW + sc * SC_ROWS
        pltpu.sync_copy(out_v, out_ref.at[pl.ds(row0, SC_ROWS)])

    pending = fire(0)
    for sc in range(NSC):
        nxt = fire(sc + 1) if sc + 1 < NSC else []
        for h in pending:
            h.wait()
        pending = nxt
        reduce_chunk(sc)


def kernel(inputs, table):
    table4 = table.reshape(table.shape[0] // GROUP, D * GROUP)
    idx3 = inputs.reshape(NW, IDX_ROWS, IDX_COLS)
    mesh = plsc.VectorSubcoreMesh(core_axis_name="c", subcore_axis_name="s")
    k = pl.kernel(
        _sc_body,
        out_type=jax.ShapeDtypeStruct((B, D), jnp.float32),
        mesh=mesh,
        scratch_types=[
            pltpu.VMEM((IDX_ROWS, IDX_COLS), jnp.int32),
            pltpu.VMEM((TPW,), jnp.int32),
            pltpu.VMEM((TPW + HALF,), jnp.int32),
            pltpu.VMEM((SC_TOK, D * GROUP), jnp.float32),
            pltpu.VMEM((SC_TOK, D * GROUP), jnp.float32),
            pltpu.VMEM((SC_ROWS, D), jnp.float32),
            pltpu.SemaphoreType.DMA,
            pltpu.SemaphoreType.DMA,
        ],
    )
    return k(table4, idx3)


# speedup vs baseline: 1.0054x; 1.0054x over previous
import jax
import jax.numpy as jnp
from jax import lax
from jax.experimental import pallas as pl
from jax.experimental.pallas import tpu as pltpu
from jax.experimental.pallas import tpu_sc as plsc

B = 4096
L = 50
D = 32
NUM_CORES = 2
NUM_SUBCORES = 16
NW = NUM_CORES * NUM_SUBCORES
BPW = B // NW
TPW = BPW * L
GROUP = 128 // D
SC_ROWS = 8
SC_TOK = SC_ROWS * L
NSC = BPW // SC_ROWS
GCH = 80
GPS = SC_TOK // GCH
HALF = 16
IDX_COLS = 64
IDX_ROWS = TPW // IDX_COLS


def _sc_body(table_ref, idx_ref, out_ref, idx_v, gidx_v, colb_v,
             buf0, buf1, out_v, sem0, sem1):
    wid = lax.axis_index("s") * NUM_CORES + lax.axis_index("c")

    pltpu.sync_copy(idx_ref.at[wid], idx_v)

    def prep(k, carry):
        j = k >> 2
        c = (k & 3) * HALF
        v = idx_v[j, pl.ds(c, HALF)]
        t = k * HALF
        gidx_v[pl.ds(t, HALF)] = lax.shift_right_logical(v, 2)
        colb_v[pl.ds(t, HALF)] = lax.shift_left(v & 3, 5)
        return carry

    lax.fori_loop(0, TPW // HALF, prep, 0)

    bufs = (buf0, buf1)
    sems = (sem0, sem1)

    def fire(sc):
        buf = bufs[sc % 2]
        sem = sems[sc % 2]
        handles = []
        for m in range(GPS):
            off = sc * SC_TOK + m * GCH
            h = pltpu.async_copy(
                table_ref.at[gidx_v.at[pl.ds(off, GCH)]],
                buf.at[pl.ds(m * GCH, GCH)],
                sem,
            )
            handles.append(h)
        return handles

    inv_l = jnp.float32(1.0 / L)

    def reduce_chunk(sc):
        buf = bufs[sc % 2]
        t0 = sc * SC_TOK

        def body(b, carry):
            p0 = b * L
            base = t0 + p0
            cvecs = [colb_v[pl.ds(base + HALF * i, HALF)]
                     for i in range((L + HALF - 1) // HALF + 1)]
            cbs = [cvecs[l // HALF][l % HALF] for l in range(L)]
            for h in range(2):
                parts = [buf[p0 + k, pl.ds(cbs[k] + h * HALF, HALF)]
                         for k in range(4)]
                for l in range(4, L):
                    parts[l % 4] = parts[l % 4] + buf[
                        p0 + l, pl.ds(cbs[l] + h * HALF, HALF)]
                s = (parts[0] + parts[1]) + (parts[2] + parts[3])
                out_v[b, pl.ds(h * HALF, HALF)] = s * inv_l
            return carry

        lax.fori_loop(0, SC_ROWS, body, 0)
        row0 = wid * BPW + sc * SC_ROWS
        pltpu.sync_copy(out_v, out_ref.at[pl.ds(row0, SC_ROWS)])

    pending = fire(0)
    for sc in range(NSC):
        nxt = fire(sc + 1) if sc + 1 < NSC else []
        for h in pending:
            h.wait()
        pending = nxt
        reduce_chunk(sc)


def kernel(inputs, table):
    table4 = table.reshape(table.shape[0] // GROUP, D * GROUP)
    idx3 = inputs.reshape(NW, IDX_ROWS, IDX_COLS)
    mesh = plsc.VectorSubcoreMesh(core_axis_name="c", subcore_axis_name="s")
    k = pl.kernel(
        _sc_body,
        out_type=jax.ShapeDtypeStruct((B, D), jnp.float32),
        mesh=mesh,
        scratch_types=[
            pltpu.VMEM((IDX_ROWS, IDX_COLS), jnp.int32),
            pltpu.VMEM((TPW,), jnp.int32),
            pltpu.VMEM((TPW + HALF,), jnp.int32),
            pltpu.VMEM((SC_TOK, D * GROUP), jnp.float32),
            pltpu.VMEM((SC_TOK, D * GROUP), jnp.float32),
            pltpu.VMEM((SC_ROWS, D), jnp.float32),
            pltpu.SemaphoreType.DMA,
            pltpu.SemaphoreType.DMA,
        ],
        compiler_params=pltpu.CompilerParams(use_tc_tiling_on_sc=True),
    )
    return k(table4, idx3)
